# baseline (speedup 1.0000x reference)
import functools

import jax
import jax.numpy as jnp
from jax import lax
from jax.experimental import pallas as pl
from jax.experimental.pallas import tpu as pltpu
from jax.experimental.pallas import tpu_sc as plsc

BATCH = 16384
EMBED = 32
PACK = 128 // EMBED
CHUNK = 128


def _transpose_body(t_ref, o_ref):
  o_ref[...] = t_ref[...].T


def _transpose_call(tab128):
  n = tab128.shape[1]
  cb = 4096
  grid = (pl.cdiv(n, cb),)
  return pl.pallas_call(
      _transpose_body,
      grid=grid,
      in_specs=[pl.BlockSpec((128, cb), lambda i: (0, i))],
      out_specs=pl.BlockSpec((cb, 128), lambda i: (i, 0)),
      out_shape=jax.ShapeDtypeStruct((n, 128), jnp.float32),
  )(tab128)


def _make_gather(nlu, nlm):
  info = plsc.get_sparse_core_info()
  nc, ns = info.num_cores, info.num_subcores
  nw = nc * ns
  b_per_w = BATCH // nw
  n_ch = b_per_w // CHUNK

  mesh = plsc.VectorSubcoreMesh(core_axis_name="c", subcore_axis_name="s")

  @functools.partial(
      pl.kernel,
      mesh=mesh,
      out_type=[
          jax.ShapeDtypeStruct((BATCH, 128), jnp.float32),
          jax.ShapeDtypeStruct((BATCH, 128), jnp.float32),
      ],
      scratch_types=[
          pltpu.VMEM((b_per_w,), jnp.int32),
          pltpu.VMEM((b_per_w,), jnp.int32),
          pltpu.VMEM((n_ch, CHUNK), jnp.int32),
          pltpu.VMEM((n_ch, CHUNK), jnp.int32),
          pltpu.VMEM((CHUNK, 128), jnp.float32),
          pltpu.VMEM((CHUNK, 128), jnp.float32),
          pltpu.VMEM((CHUNK, 128), jnp.float32),
          pltpu.VMEM((CHUNK, 128), jnp.float32),
          pltpu.SemaphoreType.DMA,
          pltpu.SemaphoreType.DMA,
      ],
  )
  def gather(uidx_hbm, midx_hbm, utab_hbm, mtab_hbm, uout_hbm, mout_hbm,
             uidx_v, midx_v, ul_v, ml_v, ubuf0, ubuf1, mbuf0, mbuf1,
             sem0, sem1):
    wid = lax.axis_index("s") * nc + lax.axis_index("c")
    base = wid * b_per_w
    pltpu.sync_copy(uidx_hbm.at[pl.ds(base, b_per_w)], uidx_v)
    pltpu.sync_copy(midx_hbm.at[pl.ds(base, b_per_w)], midx_v)

    for j in range(n_ch):
      def lines(g2, carry):
        g = j * 8 + g2
        r = uidx_v[pl.ds(g * 16, 16)]
        ul_v[j, pl.ds(g2 * 16, 16)] = lax.rem(r, jnp.int32(nlu))
        s = midx_v[pl.ds(g * 16, 16)]
        ml_v[j, pl.ds(g2 * 16, 16)] = lax.rem(s, jnp.int32(nlm))
        return carry

      lax.fori_loop(0, CHUNK // 16, lines, 0)

    sems = (sem0, sem1)
    ubufs = (ubuf0, ubuf1)
    mbufs = (mbuf0, mbuf1)

    def issue(j, slot):
      pltpu.async_copy(utab_hbm.at[ul_v.at[j]], ubufs[slot], sems[slot])
      pltpu.async_copy(mtab_hbm.at[ml_v.at[j]], mbufs[slot], sems[slot])

    def drain_out(j, slot):
      pltpu.make_async_copy(uout_hbm.at[pl.ds(0, CHUNK)], ubufs[slot],
                            sems[slot]).wait()
      pltpu.make_async_copy(mout_hbm.at[pl.ds(0, CHUNK)], mbufs[slot],
                            sems[slot]).wait()
      pltpu.sync_copy(ubufs[slot],
                      uout_hbm.at[pl.ds(base + j * CHUNK, CHUNK)])
      pltpu.sync_copy(mbufs[slot],
                      mout_hbm.at[pl.ds(base + j * CHUNK, CHUNK)])

    issue(0, 0)
    issue(1, 1)
    drain_out(0, 0)
    issue(2, 0)
    drain_out(1, 1)
    issue(3, 1)
    drain_out(2, 0)
    drain_out(3, 1)

  return gather


def _mlp_body(u_ref, m_ref, uq_ref, mq_ref, w1u_ref, w1m_ref, b1_ref,
              w2_ref, b2_ref, w3_ref, b3_ref, o_ref):
  lane_q = jax.lax.broadcasted_iota(jnp.int32, (1, 128), 1) % PACK
  um = jnp.where(lane_q == uq_ref[...], 1.0, 0.0) * u_ref[...]
  mm = jnp.where(lane_q == mq_ref[...], 1.0, 0.0) * m_ref[...]
  h1 = jnp.dot(um, w1u_ref[...], preferred_element_type=jnp.float32)
  h1 = h1 + jnp.dot(mm, w1m_ref[...], preferred_element_type=jnp.float32)
  h1 = jnp.maximum(h1 + b1_ref[...], 0.0)
  h2 = jnp.dot(h1, w2_ref[...], preferred_element_type=jnp.float32)
  h2 = jnp.maximum(h2 + b2_ref[...], 0.0)
  o_ref[...] = jnp.sum(h2 * w3_ref[...], axis=1, keepdims=True) + b3_ref[...]


def _mlp_call(u_lines, m_lines, uq, mq, W1, b1, W2, b2, W3, b3):
  bb = 2048
  grid = (BATCH // bb,)
  w1u = jnp.repeat(W1[:EMBED, :], PACK, axis=0)
  w1m = jnp.repeat(W1[EMBED:, :], PACK, axis=0)
  return pl.pallas_call(
      _mlp_body,
      grid=grid,
      in_specs=[
          pl.BlockSpec((bb, 128), lambda i: (i, 0)),
          pl.BlockSpec((bb, 128), lambda i: (i, 0)),
          pl.BlockSpec((bb, 1), lambda i: (i, 0)),
          pl.BlockSpec((bb, 1), lambda i: (i, 0)),
          pl.BlockSpec((128, 128), lambda i: (0, 0)),
          pl.BlockSpec((128, 128), lambda i: (0, 0)),
          pl.BlockSpec((1, 128), lambda i: (0, 0)),
          pl.BlockSpec((128, 64), lambda i: (0, 0)),
          pl.BlockSpec((1, 64), lambda i: (0, 0)),
          pl.BlockSpec((1, 64), lambda i: (0, 0)),
          pl.BlockSpec((1, 1), lambda i: (0, 0)),
      ],
      out_specs=pl.BlockSpec((bb, 1), lambda i: (i, 0)),
      out_shape=jax.ShapeDtypeStruct((BATCH, 1), jnp.float32),
  )(u_lines, m_lines, uq, mq, w1u, w1m, b1.reshape(1, 128), W2,
    b2.reshape(1, 64), W3.reshape(1, 64), b3.reshape(1, 1))


def kernel(user_input, movie_input, user_table, movie_table,
           W1, b1, W2, b2, W3, b3):
  nlu = user_table.shape[0] // PACK
  nlm = movie_table.shape[0] // PACK
  utab = _transpose_call(user_table.T.reshape(128, nlu))
  mtab = _transpose_call(movie_table.T.reshape(128, nlm))
  gather = _make_gather(nlu, nlm)
  u_lines, m_lines = gather(user_input, movie_input, utab, mtab)
  uq = (user_input // nlu).reshape(BATCH, 1)
  mq = (movie_input // nlm).reshape(BATCH, 1)
  return _mlp_call(u_lines, m_lines, uq, mq, W1, b1, W2, b2, W3, b3)


# speedup vs baseline: 1.0799x; 1.0799x over previous
import functools

import jax
import jax.numpy as jnp
from jax import lax
from jax.experimental import pallas as pl
from jax.experimental.pallas import tpu as pltpu
from jax.experimental.pallas import tpu_sc as plsc

BATCH = 16384
EMBED = 32
PACK = 128 // EMBED
CHUNK = 128


CB = 16384
QB = CB // PACK


def _transpose_body(t_ref, o_ref):
  for q in range(PACK):
    o_ref[:, q * EMBED:(q + 1) * EMBED] = t_ref[:, q * QB:(q + 1) * QB].T


def _transpose_call(tab_t):
  n = tab_t.shape[1]
  nb = pl.cdiv(n, CB)
  grid = (nb,)
  return pl.pallas_call(
      _transpose_body,
      grid=grid,
      in_specs=[pl.BlockSpec((EMBED, CB), lambda i: (0, i))],
      out_specs=pl.BlockSpec((QB, 128), lambda i: (i, 0)),
      out_shape=jax.ShapeDtypeStruct((nb * QB, 128), jnp.float32),
  )(tab_t)


def _make_gather(nlu, nlm):
  info = plsc.get_sparse_core_info()
  nc, ns = info.num_cores, info.num_subcores
  nw = nc * ns
  b_per_w = BATCH // nw
  n_ch = b_per_w // CHUNK

  mesh = plsc.VectorSubcoreMesh(core_axis_name="c", subcore_axis_name="s")

  @functools.partial(
      pl.kernel,
      mesh=mesh,
      out_type=[
          jax.ShapeDtypeStruct((BATCH, 128), jnp.float32),
          jax.ShapeDtypeStruct((BATCH, 128), jnp.float32),
      ],
      scratch_types=[
          pltpu.VMEM((b_per_w,), jnp.int32),
          pltpu.VMEM((b_per_w,), jnp.int32),
          pltpu.VMEM((n_ch, CHUNK), jnp.int32),
          pltpu.VMEM((n_ch, CHUNK), jnp.int32),
          pltpu.VMEM((CHUNK, 128), jnp.float32),
          pltpu.VMEM((CHUNK, 128), jnp.float32),
          pltpu.VMEM((CHUNK, 128), jnp.float32),
          pltpu.VMEM((CHUNK, 128), jnp.float32),
          pltpu.SemaphoreType.DMA,
          pltpu.SemaphoreType.DMA,
      ],
  )
  def gather(uidx_hbm, midx_hbm, utab_hbm, mtab_hbm, uout_hbm, mout_hbm,
             uidx_v, midx_v, ul_v, ml_v, ubuf0, ubuf1, mbuf0, mbuf1,
             sem0, sem1):
    wid = lax.axis_index("s") * nc + lax.axis_index("c")
    base = wid * b_per_w
    pltpu.sync_copy(uidx_hbm.at[pl.ds(base, b_per_w)], uidx_v)
    pltpu.sync_copy(midx_hbm.at[pl.ds(base, b_per_w)], midx_v)

    for j in range(n_ch):
      def lines(g2, carry):
        g = j * 8 + g2
        r = uidx_v[pl.ds(g * 16, 16)]
        ul_v[j, pl.ds(g2 * 16, 16)] = ((r >> 14) << 12) | (r & (QB - 1))
        s = midx_v[pl.ds(g * 16, 16)]
        ml_v[j, pl.ds(g2 * 16, 16)] = ((s >> 14) << 12) | (s & (QB - 1))
        return carry

      lax.fori_loop(0, CHUNK // 16, lines, 0)

    sems = (sem0, sem1)
    ubufs = (ubuf0, ubuf1)
    mbufs = (mbuf0, mbuf1)

    def issue(j, slot):
      pltpu.async_copy(utab_hbm.at[ul_v.at[j]], ubufs[slot], sems[slot])
      pltpu.async_copy(mtab_hbm.at[ml_v.at[j]], mbufs[slot], sems[slot])

    def drain_out(j, slot):
      pltpu.make_async_copy(uout_hbm.at[pl.ds(0, CHUNK)], ubufs[slot],
                            sems[slot]).wait()
      pltpu.make_async_copy(mout_hbm.at[pl.ds(0, CHUNK)], mbufs[slot],
                            sems[slot]).wait()
      pltpu.sync_copy(ubufs[slot],
                      uout_hbm.at[pl.ds(base + j * CHUNK, CHUNK)])
      pltpu.sync_copy(mbufs[slot],
                      mout_hbm.at[pl.ds(base + j * CHUNK, CHUNK)])

    issue(0, 0)
    issue(1, 1)
    drain_out(0, 0)
    issue(2, 0)
    drain_out(1, 1)
    issue(3, 1)
    drain_out(2, 0)
    drain_out(3, 1)

  return gather


def _mlp_body(u_ref, m_ref, uq_ref, mq_ref, w1u_ref, w1m_ref, b1_ref,
              w2_ref, b2_ref, w3_ref, b3_ref, o_ref):
  lane_q = jax.lax.broadcasted_iota(jnp.int32, (1, 128), 1) // EMBED
  um = jnp.where(lane_q == uq_ref[...], 1.0, 0.0) * u_ref[...]
  mm = jnp.where(lane_q == mq_ref[...], 1.0, 0.0) * m_ref[...]
  h1 = jnp.dot(um, w1u_ref[...], preferred_element_type=jnp.float32)
  h1 = h1 + jnp.dot(mm, w1m_ref[...], preferred_element_type=jnp.float32)
  h1 = jnp.maximum(h1 + b1_ref[...], 0.0)
  h2 = jnp.dot(h1, w2_ref[...], preferred_element_type=jnp.float32)
  h2 = jnp.maximum(h2 + b2_ref[...], 0.0)
  o_ref[...] = jnp.sum(h2 * w3_ref[...], axis=1, keepdims=True) + b3_ref[...]


def _mlp_call(u_lines, m_lines, uq, mq, W1, b1, W2, b2, W3, b3):
  bb = 2048
  grid = (BATCH // bb,)
  w1u = jnp.tile(W1[:EMBED, :], (PACK, 1))
  w1m = jnp.tile(W1[EMBED:, :], (PACK, 1))
  return pl.pallas_call(
      _mlp_body,
      grid=grid,
      in_specs=[
          pl.BlockSpec((bb, 128), lambda i: (i, 0)),
          pl.BlockSpec((bb, 128), lambda i: (i, 0)),
          pl.BlockSpec((bb, 1), lambda i: (i, 0)),
          pl.BlockSpec((bb, 1), lambda i: (i, 0)),
          pl.BlockSpec((128, 128), lambda i: (0, 0)),
          pl.BlockSpec((128, 128), lambda i: (0, 0)),
          pl.BlockSpec((1, 128), lambda i: (0, 0)),
          pl.BlockSpec((128, 64), lambda i: (0, 0)),
          pl.BlockSpec((1, 64), lambda i: (0, 0)),
          pl.BlockSpec((1, 64), lambda i: (0, 0)),
          pl.BlockSpec((1, 1), lambda i: (0, 0)),
      ],
      out_specs=pl.BlockSpec((bb, 1), lambda i: (i, 0)),
      out_shape=jax.ShapeDtypeStruct((BATCH, 1), jnp.float32),
  )(u_lines, m_lines, uq, mq, w1u, w1m, b1.reshape(1, 128), W2,
    b2.reshape(1, 64), W3.reshape(1, 64), b3.reshape(1, 1))


def kernel(user_input, movie_input, user_table, movie_table,
           W1, b1, W2, b2, W3, b3):
  utab = _transpose_call(user_table.T)
  mtab = _transpose_call(movie_table.T)
  gather = _make_gather(utab.shape[0], mtab.shape[0])
  u_lines, m_lines = gather(user_input, movie_input, utab, mtab)
  uq = ((user_input >> 12) & 3).reshape(BATCH, 1)
  mq = ((movie_input >> 12) & 3).reshape(BATCH, 1)
  return _mlp_call(u_lines, m_lines, uq, mq, W1, b1, W2, b2, W3, b3)


# speedup vs baseline: 1.1164x; 1.0338x over previous
import functools

import jax
import jax.numpy as jnp
from jax import lax
from jax.experimental import pallas as pl
from jax.experimental.pallas import tpu as pltpu
from jax.experimental.pallas import tpu_sc as plsc

BATCH = 16384
EMBED = 32
GROUP = 8
CH = 16


def _transpose_body(t_ref, o_ref):
  o_ref[...] = t_ref[...].T


def _transpose_call(tab_t):
  n = tab_t.shape[1]
  cb = 32768
  grid = (pl.cdiv(n, cb),)
  return pl.pallas_call(
      _transpose_body,
      grid=grid,
      in_specs=[pl.BlockSpec((EMBED, cb), lambda i: (0, i))],
      out_specs=pl.BlockSpec((cb, EMBED), lambda i: (i, 0)),
      out_shape=jax.ShapeDtypeStruct((n, EMBED), jnp.float32),
  )(tab_t)


def _make_gather():
  info = plsc.get_sparse_core_info()
  nc, ns = info.num_cores, info.num_subcores
  nw = nc * ns
  b_per_w = BATCH // nw
  n_ch = b_per_w // CH

  mesh = plsc.VectorSubcoreMesh(core_axis_name="c", subcore_axis_name="s")

  @functools.partial(
      pl.kernel,
      mesh=mesh,
      out_type=[
          jax.ShapeDtypeStruct((BATCH, EMBED), jnp.float32),
          jax.ShapeDtypeStruct((BATCH, EMBED), jnp.float32),
      ],
      scratch_types=[
          pltpu.VMEM((b_per_w,), jnp.int32),
          pltpu.VMEM((b_per_w,), jnp.int32),
          pltpu.VMEM((2, CH * GROUP, EMBED), jnp.float32),
          pltpu.VMEM((2, CH * GROUP, EMBED), jnp.float32),
          pltpu.VMEM((CH, EMBED), jnp.float32),
          pltpu.VMEM((CH, EMBED), jnp.float32),
          pltpu.SemaphoreType.DMA,
          pltpu.SemaphoreType.DMA,
      ],
  )
  def gather(uidx_hbm, midx_hbm, utab_hbm, mtab_hbm, uout_hbm, mout_hbm,
             uidx_v, midx_v, ubuf_v, mbuf_v, uo_v, mo_v, sem0, sem1):
    wid = lax.axis_index("s") * nc + lax.axis_index("c")
    base = wid * b_per_w
    pltpu.sync_copy(uidx_hbm.at[pl.ds(base, b_per_w)], uidx_v)
    pltpu.sync_copy(midx_hbm.at[pl.ds(base, b_per_w)], midx_v)
    sems = (sem0, sem1)

    def issue(c, slot):
      sem = sems[slot]
      uvec = (uidx_v[pl.ds(c * CH, CH)] >> 3) * GROUP
      mvec = (midx_v[pl.ds(c * CH, CH)] >> 3) * GROUP
      ub = ubuf_v.at[slot]
      mb = mbuf_v.at[slot]
      for k in range(CH):
        rr = pl.multiple_of(uvec[k], GROUP)
        pltpu.async_copy(utab_hbm.at[pl.ds(rr, GROUP)],
                         ub.at[pl.ds(k * GROUP, GROUP)], sem)
        ss = pl.multiple_of(mvec[k], GROUP)
        pltpu.async_copy(mtab_hbm.at[pl.ds(ss, GROUP)],
                         mb.at[pl.ds(k * GROUP, GROUP)], sem)

    def drain_select_out(c, slot):
      sem = sems[slot]
      pltpu.make_async_copy(uout_hbm.at[pl.ds(0, CH * GROUP)],
                            ubuf_v.at[slot], sem).wait()
      pltpu.make_async_copy(mout_hbm.at[pl.ds(0, CH * GROUP)],
                            mbuf_v.at[slot], sem).wait()
      uq = (uidx_v[pl.ds(c * CH, CH)] & (GROUP - 1))
      mq = (midx_v[pl.ds(c * CH, CH)] & (GROUP - 1))
      for k in range(CH):
        ur = k * GROUP + uq[k]
        uo_v[k, pl.ds(0, 16)] = ubuf_v[slot, ur, pl.ds(0, 16)]
        uo_v[k, pl.ds(16, 16)] = ubuf_v[slot, ur, pl.ds(16, 16)]
        mr = k * GROUP + mq[k]
        mo_v[k, pl.ds(0, 16)] = mbuf_v[slot, mr, pl.ds(0, 16)]
        mo_v[k, pl.ds(16, 16)] = mbuf_v[slot, mr, pl.ds(16, 16)]
      pltpu.sync_copy(uo_v, uout_hbm.at[pl.ds(base + c * CH, CH)])
      pltpu.sync_copy(mo_v, mout_hbm.at[pl.ds(base + c * CH, CH)])

    issue(0, 0)

    def body(c2, carry):
      c = c2 * 2
      issue(c + 1, 1)
      drain_select_out(c, 0)

      @pl.when(c2 + 1 < n_ch // 2)
      def _():
        issue(c + 2, 0)

      drain_select_out(c + 1, 1)
      return carry

    lax.fori_loop(0, n_ch // 2, body, 0)

  return gather


def _mlp_body(u_ref, m_ref, w1_ref, b1_ref, w2_ref, b2_ref, w3_ref, b3_ref,
              o_ref):
  h1 = jnp.dot(u_ref[...], w1_ref[0:EMBED, :],
               preferred_element_type=jnp.float32)
  h1 = h1 + jnp.dot(m_ref[...], w1_ref[EMBED:2 * EMBED, :],
                    preferred_element_type=jnp.float32)
  h1 = jnp.maximum(h1 + b1_ref[...], 0.0)
  h2 = jnp.dot(h1, w2_ref[...], preferred_element_type=jnp.float32)
  h2 = jnp.maximum(h2 + b2_ref[...], 0.0)
  o_ref[...] = jnp.sum(h2 * w3_ref[...], axis=1, keepdims=True) + b3_ref[...]


def _mlp_call(u_emb, m_emb, W1, b1, W2, b2, W3, b3):
  bb = 2048
  grid = (BATCH // bb,)
  return pl.pallas_call(
      _mlp_body,
      grid=grid,
      in_specs=[
          pl.BlockSpec((bb, EMBED), lambda i: (i, 0)),
          pl.BlockSpec((bb, EMBED), lambda i: (i, 0)),
          pl.BlockSpec((2 * EMBED, 128), lambda i: (0, 0)),
          pl.BlockSpec((1, 128), lambda i: (0, 0)),
          pl.BlockSpec((128, 64), lambda i: (0, 0)),
          pl.BlockSpec((1, 64), lambda i: (0, 0)),
          pl.BlockSpec((1, 64), lambda i: (0, 0)),
          pl.BlockSpec((1, 1), lambda i: (0, 0)),
      ],
      out_specs=pl.BlockSpec((bb, 1), lambda i: (i, 0)),
      out_shape=jax.ShapeDtypeStruct((BATCH, 1), jnp.float32),
  )(u_emb, m_emb, W1, b1.reshape(1, 128), W2, b2.reshape(1, 64),
    W3.reshape(1, 64), b3.reshape(1, 1))


def kernel(user_input, movie_input, user_table, movie_table,
           W1, b1, W2, b2, W3, b3):
  utab = _transpose_call(user_table.T)
  mtab = _transpose_call(movie_table.T)
  gather = _make_gather()
  u_emb, m_emb = gather(user_input, movie_input, utab, mtab)
  return _mlp_call(u_emb, m_emb, W1, b1, W2, b2, W3, b3)
